# Initial kernel scaffold; baseline (speedup 1.0000x reference)
import jax
import jax.numpy as jnp
from jax.experimental import pallas as pl


def kernel(edge_index, emb_weight):
    raise NotImplementedError("write your pallas kernel here")


# speedup vs baseline: 1.4341x; 1.4341x over previous
import functools

import jax
import jax.numpy as jnp
from jax import lax
from jax.experimental import pallas as pl
from jax.experimental.pallas import tpu as pltpu
from jax.experimental.pallas import tpu_sc as plsc

_E = 3_200_000
_D = 16
_NC = 2
_NS = 16
_NW = _NC * _NS
_PER_W = _E // _NW
_B = 2_000
_NBLK = _PER_W // _B
_L = 16


def _sc_body(edge_hbm, table_hbm, out_hbm, i0_v, i1_v, codes_v, rows_v, sem):
    wid = lax.axis_index("s") * _NC + lax.axis_index("c")
    start = wid * _PER_W

    def block(b, carry):
        base = start + b * _B
        pltpu.sync_copy(edge_hbm.at[pl.ds(base, _B)], i0_v)
        pltpu.sync_copy(edge_hbm.at[pl.ds(_E + base, _B)], i1_v)

        def grp(j, c):
            o = j * _L
            a = jnp.abs(i0_v[pl.ds(o, _L)] - i1_v[pl.ds(o, _L)])
            t = jnp.maximum(a - 1, 0).astype(jnp.float32)
            bits = lax.bitcast_convert_type(t, jnp.int32)
            codes_v[pl.ds(o, _L)] = jnp.maximum((bits >> 23) - 126, 0)
            return c

        lax.fori_loop(0, _B // _L, grp, 0)
        pltpu.async_copy(table_hbm.at[codes_v], rows_v, sem).wait()
        pltpu.sync_copy(rows_v, out_hbm.at[pl.ds(base, _B)])
        return carry

    lax.fori_loop(0, _NBLK, block, 0)


def kernel(edge_index, emb_weight):
    run = pl.kernel(
        _sc_body,
        out_type=jax.ShapeDtypeStruct((_E, _D), jnp.float32),
        mesh=plsc.VectorSubcoreMesh(core_axis_name="c", subcore_axis_name="s"),
        compiler_params=pltpu.CompilerParams(use_tc_tiling_on_sc=False),
        scratch_types=[
            pltpu.VMEM((_B,), jnp.int32),
            pltpu.VMEM((_B,), jnp.int32),
            pltpu.VMEM((_B,), jnp.int32),
            pltpu.VMEM((_B, _D), jnp.float32),
            pltpu.SemaphoreType.DMA,
        ],
    )
    return run(edge_index.reshape(-1), emb_weight)


# speedup vs baseline: 7.6839x; 5.3578x over previous
import functools

import jax
import jax.numpy as jnp
from jax import lax
from jax.experimental import pallas as pl
from jax.experimental.pallas import tpu as pltpu
from jax.experimental.pallas import tpu_sc as plsc

_E = 3_200_000
_D = 16
_NC = 2
_NS = 16
_NW = _NC * _NS
_PER_W = _E // _NW
_B = 2_000
_NBLK = _PER_W // _B
_L = 16


def _sc_body(edge_hbm, table_hbm, out_hbm, table_v, i0_v, i1_v, codes_v,
             rows_v, sem):
    sid = lax.axis_index("s")
    wid = sid * _NC + lax.axis_index("c")
    start = wid * _PER_W
    @pl.when(sid == 0)
    def _stage():
        pltpu.sync_copy(table_hbm, table_v)

    plsc.subcore_barrier()

    def block(b, carry):
        base = start + b * _B
        pltpu.sync_copy(edge_hbm.at[pl.ds(base, _B)], i0_v)
        pltpu.sync_copy(edge_hbm.at[pl.ds(_E + base, _B)], i1_v)

        def grp(j, c):
            o = j * _L
            a = jnp.abs(i0_v[pl.ds(o, _L)] - i1_v[pl.ds(o, _L)])
            t = jnp.maximum(a - 1, 0).astype(jnp.float32)
            bits = lax.bitcast_convert_type(t, jnp.int32)
            codes_v[pl.ds(o, _L)] = jnp.maximum((bits >> 23) - 126, 0)
            return c

        lax.fori_loop(0, _B // _L, grp, 0, unroll=4)
        pltpu.async_copy(table_v.at[codes_v], rows_v, sem).wait()
        pltpu.sync_copy(rows_v, out_hbm.at[pl.ds(base, _B)])
        return carry

    lax.fori_loop(0, _NBLK, block, 0)


def kernel(edge_index, emb_weight):
    run = pl.kernel(
        _sc_body,
        out_type=jax.ShapeDtypeStruct((_E, _D), jnp.float32),
        mesh=plsc.VectorSubcoreMesh(core_axis_name="c", subcore_axis_name="s"),
        compiler_params=pltpu.CompilerParams(use_tc_tiling_on_sc=False),
        scratch_types=[
            pltpu.VMEM_SHARED((18, _D), jnp.float32),
            pltpu.VMEM((_B,), jnp.int32),
            pltpu.VMEM((_B,), jnp.int32),
            pltpu.VMEM((_B,), jnp.int32),
            pltpu.VMEM((_B, _D), jnp.float32),
            pltpu.SemaphoreType.DMA,
        ],
    )
    return run(edge_index.reshape(-1), emb_weight)


# speedup vs baseline: 8.1981x; 1.0669x over previous
import functools

import jax
import jax.numpy as jnp
from jax import lax
from jax.experimental import pallas as pl
from jax.experimental.pallas import tpu as pltpu
from jax.experimental.pallas import tpu_sc as plsc

_E = 3_200_000
_D = 16
_NC = 2
_NS = 16
_NW = _NC * _NS
_PER_W = _E // _NW
_B = 2_000
_NBLK = _PER_W // _B
_L = 16


def _sc_body(edge_hbm, table_hbm, out_hbm, table_v, i0_v, i1_v, codes_v,
             rows_v, sem):
    sid = lax.axis_index("s")
    wid = sid * _NC + lax.axis_index("c")
    start = wid * _PER_W
    pltpu.sync_copy(table_hbm, table_v.at[sid])

    def block(b, carry):
        base = start + b * _B
        pltpu.sync_copy(edge_hbm.at[pl.ds(base, _B)], i0_v)
        pltpu.sync_copy(edge_hbm.at[pl.ds(_E + base, _B)], i1_v)

        def grp(j, c):
            o = j * _L
            a = jnp.abs(i0_v[pl.ds(o, _L)] - i1_v[pl.ds(o, _L)])
            t = jnp.maximum(a - 1, 0).astype(jnp.float32)
            bits = lax.bitcast_convert_type(t, jnp.int32)
            codes_v[pl.ds(o, _L)] = jnp.maximum((bits >> 23) - 126, 0)
            return c

        lax.fori_loop(0, _B // _L, grp, 0, unroll=4)
        pltpu.async_copy(table_v.at[sid].at[codes_v], rows_v, sem).wait()
        pltpu.sync_copy(rows_v, out_hbm.at[pl.ds(base, _B)])
        return carry

    lax.fori_loop(0, _NBLK, block, 0)


def kernel(edge_index, emb_weight):
    run = pl.kernel(
        _sc_body,
        out_type=jax.ShapeDtypeStruct((_E, _D), jnp.float32),
        mesh=plsc.VectorSubcoreMesh(core_axis_name="c", subcore_axis_name="s"),
        compiler_params=pltpu.CompilerParams(use_tc_tiling_on_sc=False),
        scratch_types=[
            pltpu.VMEM_SHARED((_NS, 18, _D), jnp.float32),
            pltpu.VMEM((_B,), jnp.int32),
            pltpu.VMEM((_B,), jnp.int32),
            pltpu.VMEM((_B,), jnp.int32),
            pltpu.VMEM((_B, _D), jnp.float32),
            pltpu.SemaphoreType.DMA,
        ],
    )
    return run(edge_index.reshape(-1), emb_weight)


# speedup vs baseline: 21.3124x; 2.5997x over previous
import functools

import jax
import jax.numpy as jnp
from jax import lax
from jax.experimental import pallas as pl
from jax.experimental.pallas import tpu as pltpu
from jax.experimental.pallas import tpu_sc as plsc

_E = 3_200_000
_D = 16
_NC = 2
_NS = 16
_NW = _NC * _NS
_B = 2_560
_NBLK = _E // _B
_L = 16


def _sc_body(edge_hbm, table_hbm, out_hbm, table_v, i0_v, i1_v, rows_v, sem):
    wid = lax.axis_index("s") * _NC + lax.axis_index("c")
    pltpu.sync_copy(table_hbm, table_v)

    def block(i, carry):
        g = wid + i * _NW
        base = g * _B
        pltpu.sync_copy(edge_hbm.at[pl.ds(base, _B)], i0_v)
        pltpu.sync_copy(edge_hbm.at[pl.ds(_E + base, _B)], i1_v)

        def grp(j, c):
            o = j * _L
            a = jnp.abs(i0_v[pl.ds(o, _L)] - i1_v[pl.ds(o, _L)])
            t = jnp.maximum(a - 1, 0).astype(jnp.float32)
            bits = lax.bitcast_convert_type(t, jnp.int32)
            codes = jnp.maximum((bits >> 23) - 126, 0)
            for d in range(_D):
                rows_v[d, pl.ds(o, _L)] = plsc.load_gather(
                    table_v, [codes + (d * 128)])
            return c

        lax.fori_loop(0, _B // _L, grp, 0)
        pltpu.sync_copy(rows_v, out_hbm.at[:, pl.ds(base, _B)])
        return carry

    lax.fori_loop(0, _NBLK // _NW + (wid < _NBLK % _NW), block, 0)


def kernel(edge_index, emb_weight):
    table_t = jnp.zeros((_D, 128), jnp.float32).at[:, :18].set(emb_weight.T)
    run = pl.kernel(
        _sc_body,
        out_type=jax.ShapeDtypeStruct((_D, _E), jnp.float32),
        mesh=plsc.VectorSubcoreMesh(core_axis_name="c", subcore_axis_name="s"),
        compiler_params=pltpu.CompilerParams(
            use_tc_tiling_on_sc=True, needs_layout_passes=False),
        scratch_types=[
            pltpu.VMEM((_D * 128,), jnp.float32),
            pltpu.VMEM((_B,), jnp.int32),
            pltpu.VMEM((_B,), jnp.int32),
            pltpu.VMEM((_D, _B), jnp.float32),
            pltpu.SemaphoreType.DMA,
        ],
    )
    out_t = run(edge_index.reshape(-1), table_t.reshape(-1))
    return out_t.T


# speedup vs baseline: 59.4243x; 2.7882x over previous
import functools

import jax
import jax.numpy as jnp
from jax import lax
from jax.experimental import pallas as pl
from jax.experimental.pallas import tpu as pltpu
from jax.experimental.pallas import tpu_sc as plsc

_E = 3_200_000
_D = 16
_NC = 2
_NS = 16
_NW = _NC * _NS
_B = 2_560
_NBLK = _E // _B
_L = 16


def _sc_body(edge_hbm, table_hbm, out_hbm, table_v, i0_v, i1_v, rows_v, sem):
    wid = lax.axis_index("s") * _NC + lax.axis_index("c")
    pltpu.sync_copy(table_hbm, table_v)

    def block(i, carry):
        g = wid + i * _NW
        base = g * _B
        pltpu.sync_copy(edge_hbm.at[pl.ds(base, _B)], i0_v)
        pltpu.sync_copy(edge_hbm.at[pl.ds(_E + base, _B)], i1_v)

        @plsc.parallel_loop(0, _B // _L, 1, unroll=2)
        def grp(j):
            o = j * _L
            a = jnp.abs(i0_v[pl.ds(o, _L)] - i1_v[pl.ds(o, _L)])
            t = jnp.maximum(a - 1, 0).astype(jnp.float32)
            bits = lax.bitcast_convert_type(t, jnp.int32)
            codes = jnp.maximum((bits >> 23) - 126, 0)
            for d in range(_D):
                rows_v[d, pl.ds(o, _L)] = plsc.load_gather(
                    table_v, [codes + (d * 128)])
        pltpu.sync_copy(rows_v, out_hbm.at[:, pl.ds(base, _B)])
        return carry

    lax.fori_loop(0, _NBLK // _NW + (wid < _NBLK % _NW), block, 0)


def kernel(edge_index, emb_weight):
    table_t = jnp.zeros((_D, 128), jnp.float32).at[:, :18].set(emb_weight.T)
    run = pl.kernel(
        _sc_body,
        out_type=jax.ShapeDtypeStruct((_D, _E), jnp.float32),
        mesh=plsc.VectorSubcoreMesh(core_axis_name="c", subcore_axis_name="s"),
        compiler_params=pltpu.CompilerParams(
            use_tc_tiling_on_sc=True, needs_layout_passes=False),
        scratch_types=[
            pltpu.VMEM((_D * 128,), jnp.float32),
            pltpu.VMEM((_B,), jnp.int32),
            pltpu.VMEM((_B,), jnp.int32),
            pltpu.VMEM((_D, _B), jnp.float32),
            pltpu.SemaphoreType.DMA,
        ],
    )
    out_t = run(edge_index.reshape(-1), table_t.reshape(-1))
    return out_t.T
